# baseline (speedup 1.0000x reference)
import functools

import jax
import jax.numpy as jnp
from jax import lax
from jax.experimental import pallas as pl
from jax.experimental.pallas import tpu as pltpu
from jax.experimental.pallas import tpu_sc as plsc

N_TX, N_U, N_M = 100000, 50000, 10000
E = 300000
D_IN, H = 128, 64

NC, NS = 2, 16
BLK = 1920
BPT = 10
SLICE = BLK * BPT
E_PAD = SLICE * NS + BLK
S = 6
CS = S * 128
RANGE = 12800
STRIPE = 808
ALLOC = STRIPE * NS
RANGE_D = 5000
STRIPE_D = 320
ALLOC_D = STRIPE_D * NS
NW = 32
BLK_P = 960
WSLICE = BLK_P * 10
PBUF = 1008
PDUMP = 992
PAPP = 960
PCAP = 10496
CHUNK = 768


def _sc_partition_kernel(types):
    n_e = 2 * len(types)

    def body(*refs):
        edges = refs[:n_e]
        outs = refs[n_e:n_e + 3 * len(types)]
        (ebs0, ebd0, ebs1, ebd1, bb_src, bb_dst, tb_src, tb_dst, gvbuf,
         sem_e, sem_a) = refs[n_e + 3 * len(types):]

        c = lax.axis_index("c")
        s = lax.axis_index("s")
        w = s * 2 + c
        zero16i = jnp.zeros((16,), jnp.int32)
        lanes = lax.iota(jnp.int32, 16)

        def wait_e():
            pltpu.make_async_copy(
                edges[0].at[pl.ds(0, BLK_P)], ebs0, sem_e).wait()

        for t_i, (src_i, dst_i, R, RR, trashv) in enumerate(types):
            src_h, dst_h = edges[src_i], edges[dst_i]
            ps, pd, pg = outs[3 * t_i], outs[3 * t_i + 1], outs[3 * t_i + 2]
            trash16 = jnp.full((16,), trashv, jnp.int32)

            def _tb(i, _):
                tb_src[pl.ds(i * 16, 16)] = zero16i
                tb_dst[pl.ds(i * 16, 16)] = trash16
                return 0
            lax.fori_loop(0, CHUNK // 16, _tb, 0)

            ebase = w * WSLICE
            sbase = w * (R * PCAP)

            def stage(b, bs, bd):
                pltpu.async_copy(
                    src_h.at[pl.ds(ebase + b * BLK_P, BLK_P)], bs, sem_e)
                pltpu.async_copy(
                    dst_h.at[pl.ds(ebase + b * BLK_P, BLK_P)], bd, sem_e)

            def wait_a():
                pltpu.make_async_copy(
                    bb_src.at[0, 0, pl.ds(0, PAPP)],
                    ps.at[pl.ds(0, PAPP)], sem_a).wait()

            def filter_block(P, ebs, ebd, nf):
                def it(i, nf_t):
                    nf = list(nf_t)
                    d = ebd[pl.ds(i * 16, 16)]
                    sv = ebs[pl.ds(i * 16, 16)]
                    for b in range(R):
                        m = (d >= b * RR) & (d < (b + 1) * RR)
                        cum = jnp.cumsum(m.astype(jnp.int32))
                        pos = jnp.where(m, nf[b] + cum - 1, PDUMP)
                        plsc.store_scatter(bb_dst.at[P, b], [pos], d - b * RR)
                        plsc.store_scatter(bb_src.at[P, b], [pos], sv)
                        nf[b] = nf[b] + jnp.max(cum)
                    return tuple(nf)
                return lax.fori_loop(0, BLK_P // 16, it, tuple(nf))

            def block_end(P, nf, off, drain_cond):
                if drain_cond is None:
                    for _ in range(2 * R):
                        wait_a()
                else:
                    @pl.when(drain_cond)
                    def _():
                        for _ in range(2 * R):
                            wait_a()
                for b in range(R):
                    oa = sbase + b * PCAP + pl.multiple_of(off[b], 8)
                    pltpu.async_copy(
                        bb_src.at[P, b, pl.ds(0, PAPP)],
                        ps.at[pl.ds(oa, PAPP)], sem_a)
                    pltpu.async_copy(
                        bb_dst.at[P, b, pl.ds(0, PAPP)],
                        pd.at[pl.ds(oa, PAPP)], sem_a)
                nf2, off2 = [], []
                for b in range(R):
                    rd = jnp.bitwise_and(nf[b], -8)
                    v = bb_src[P, b, pl.ds(rd, 16)]
                    bb_src[1 - P, b, pl.ds(0, 16)] = v
                    u = bb_dst[P, b, pl.ds(rd, 16)]
                    bb_dst[1 - P, b, pl.ds(0, 16)] = u
                    off2.append(off[b] + rd)
                    nf2.append(nf[b] - rd)
                return nf2, off2

            stage(0, ebs0, ebd0)

            def outer(o, carry):
                nf = list(carry[:R])
                off = list(carry[R:])
                wait_e()
                wait_e()
                stage(2 * o + 1, ebs1, ebd1)
                nf = list(filter_block(0, ebs0, ebd0, nf))
                nf, off = block_end(0, nf, off, o > 0)
                wait_e()
                wait_e()
                stage(2 * o + 2, ebs0, ebd0)
                nf = list(filter_block(1, ebs1, ebd1, nf))
                nf, off = block_end(1, nf, off, None)
                return tuple(nf) + tuple(off)

            zero = jnp.zeros((), jnp.int32)
            carry = lax.fori_loop(0, 5, outer, (zero,) * (2 * R))
            nf = list(carry[:R])
            off = list(carry[R:])
            for _ in range(2 * R):
                wait_a()
            gv = jnp.zeros((16,), jnp.int32)
            for b in range(R):
                bb_src[0, b, pl.ds(nf[b], 16)] = zero16i
                bb_dst[0, b, pl.ds(nf[b], 16)] = trash16
                oa = sbase + b * PCAP + pl.multiple_of(off[b], 8)
                pltpu.async_copy(
                    bb_src.at[0, b, pl.ds(0, 16)],
                    ps.at[pl.ds(oa, 16)], sem_a)
                pltpu.async_copy(
                    bb_dst.at[0, b, pl.ds(0, 16)],
                    pd.at[pl.ds(oa, 16)], sem_a)
                pltpu.async_copy(
                    tb_src, ps.at[pl.ds(oa + 16, CHUNK)], sem_a)
                pltpu.async_copy(
                    tb_dst, pd.at[pl.ds(oa + 16, CHUNK)], sem_a)
                ntot = off[b] + nf[b]
                gv = jnp.where(lanes == b, (ntot + CHUNK - 1) // CHUNK, gv)
            gvbuf[pl.ds(0, 16)] = gv
            for b in range(R):
                pltpu.make_async_copy(
                    bb_src.at[0, b, pl.ds(0, 16)],
                    ps.at[pl.ds(0, 16)], sem_a).wait()
                pltpu.make_async_copy(
                    bb_dst.at[0, b, pl.ds(0, 16)],
                    pd.at[pl.ds(0, 16)], sem_a).wait()
                pltpu.make_async_copy(
                    tb_src, ps.at[pl.ds(0, CHUNK)], sem_a).wait()
                pltpu.make_async_copy(
                    tb_dst, pd.at[pl.ds(0, CHUNK)], sem_a).wait()
            pltpu.sync_copy(gvbuf, pg.at[pl.ds(w * 16, 16)])
            wait_e()
            wait_e()

    return body


def _make_sc_partition():
    i32 = jnp.int32
    types = [
        (0, 1, 8, RANGE, ALLOC - 1),
        (2, 3, 8, RANGE, ALLOC - 1),
        (4, 5, 4, RANGE, ALLOC - 1),
        (6, 7, 2, RANGE_D, ALLOC_D - 1),
    ]
    body = _sc_partition_kernel(types)
    out_type = []
    for (_, _, R, _, _) in types:
        out_type += [jax.ShapeDtypeStruct((NW * R * PCAP,), i32),
                     jax.ShapeDtypeStruct((NW * R * PCAP,), i32),
                     jax.ShapeDtypeStruct((NW * 16,), i32)]
    return pl.kernel(
        body,
        out_type=out_type,
        mesh=plsc.VectorSubcoreMesh(core_axis_name="c", subcore_axis_name="s"),
        scratch_types=[
            pltpu.VMEM((BLK_P,), i32),
            pltpu.VMEM((BLK_P,), i32),
            pltpu.VMEM((BLK_P,), i32),
            pltpu.VMEM((BLK_P,), i32),
            pltpu.VMEM((2, 8, PBUF), i32),
            pltpu.VMEM((2, 8, PBUF), i32),
            pltpu.VMEM((CHUNK,), i32),
            pltpu.VMEM((CHUNK,), i32),
            pltpu.VMEM((16,), i32),
            pltpu.SemaphoreType.DMA,
            pltpu.SemaphoreType.DMA,
        ],
        name="sc_sage_partition",
        compiler_params=pltpu.CompilerParams(
            needs_layout_passes=False, use_tc_tiling_on_sc=False),
    )


def _sc_stream_kernel(phases, do_cnt, num_tables):
    n_out = len(phases) * (2 if do_cnt else 1)

    def body(*refs):
        nt = num_tables + 2
        tabs = refs[:num_tables]
        z2d, z1d = refs[num_tables:nt]
        nparts = 3 * len(phases)
        parts = refs[nt:nt + nparts]
        outs = refs[nt + nparts:nt + nparts + n_out]
        (acc_sp, cnt_sp, pbs, pbd, di, rows, gvbuf, ones,
         sem_g, sem_s, sem_c) = refs[nt + nparts + n_out:]

        c = lax.axis_index("c")
        s = lax.axis_index("s")
        lanes = lax.iota(jnp.int32, 16)
        one16f = jnp.ones((16,), jnp.float32)
        for u in range(8):
            ones[pl.ds(u * 16, 16)] = one16f

        def drain_batches(use_cnt):
            for _ in range(S):
                pltpu.make_async_copy(
                    rows.at[0], acc_sp.at[di.at[0]], sem_s).wait()
            if use_cnt:
                for _ in range(S):
                    pltpu.make_async_copy(
                        ones, cnt_sp.at[di.at[0]], sem_c).wait()

        out_i = 0
        for (tab_i, part_i, R, range_rows, stripe, alloc) in phases:
            table = tabs[tab_i]
            ps = parts[3 * part_i]
            pd = parts[3 * part_i + 1]
            pg = parts[3 * part_i + 2]
            acc_out = outs[out_i]
            cnt_out = outs[out_i + 1] if do_cnt else None
            out_i += 2 if do_cnt else 1
            half = R // 2

            def stream_region(wsrc, r, use_cnt):
                pltpu.sync_copy(pg.at[pl.ds(wsrc * 16, 16)], gvbuf)
                gv = gvbuf[pl.ds(0, 16)]
                ng = jnp.minimum(jnp.max(jnp.where(lanes == r, gv, 0)),
                                 PCAP // CHUNK)
                base = (wsrc * R + r) * PCAP

                nsrc = table.shape[0]

                def chunk(j, _):
                    @pl.when(j > 0)
                    def _():
                        drain_batches(use_cnt)
                    pltpu.sync_copy(ps.at[pl.ds(base + j * CHUNK, CHUNK)], pbs)
                    pltpu.sync_copy(pd.at[pl.ds(base + j * CHUNK, CHUNK)], pbd)

                    def clampit(i, _):
                        v = pbs[pl.ds(i * 16, 16)]
                        pbs[pl.ds(i * 16, 16)] = jnp.clip(v, 0, nsrc - 1)
                        u = pbd[pl.ds(i * 16, 16)]
                        pbd[pl.ds(i * 16, 16)] = jnp.clip(u, 0, alloc - 1)
                        return 0
                    lax.fori_loop(0, CHUNK // 16, clampit, 0)
                    for k in range(S):
                        for u in range(8):
                            di[k, pl.ds(u * 16, 16)] = pbd[
                                pl.ds(k * 128 + u * 16, 16)]
                    for k in range(S):
                        pltpu.async_copy(
                            table.at[pbs.at[pl.ds(k * 128, 128)]],
                            rows.at[k], sem_g)
                    for k in range(S):
                        pltpu.make_async_copy(
                            table.at[pbs.at[pl.ds(k * 128, 128)]],
                            rows.at[k], sem_g).wait()
                    for k in range(S):
                        pltpu.async_copy(
                            rows.at[k], acc_sp.at[di.at[k]], sem_s, add=True)
                        if use_cnt:
                            pltpu.async_copy(
                                ones, cnt_sp.at[di.at[k]], sem_c, add=True)
                    return 0

                lax.fori_loop(0, ng, chunk, 0)

                @pl.when(ng > 0)
                def _():
                    drain_batches(use_cnt)

            def range_body(q, _):
                r = c * half + q
                plsc.subcore_barrier()
                pltpu.sync_copy(
                    z2d.at[pl.ds(s * stripe, stripe)],
                    acc_sp.at[pl.ds(s * stripe, stripe)])
                if do_cnt:
                    pltpu.sync_copy(
                        z1d.at[pl.ds(s * stripe, stripe)],
                        cnt_sp.at[pl.ds(s * stripe, stripe)])
                plsc.subcore_barrier()
                stream_region(s * 2, r, do_cnt)
                stream_region(s * 2 + 1, r, do_cnt)
                plsc.subcore_barrier()
                pltpu.sync_copy(
                    acc_sp.at[pl.ds(s * stripe, stripe)],
                    acc_out.at[r, pl.ds(s * stripe, stripe)])
                if do_cnt:
                    pltpu.sync_copy(
                        cnt_sp.at[pl.ds(s * stripe, stripe)],
                        cnt_out.at[pl.ds(r * alloc + s * stripe, stripe)])
                return 0

            lax.fori_loop(0, half, range_body, 0)

    return body


def _stream_scratch():
    f32, i32 = jnp.float32, jnp.int32
    return [
        pltpu.VMEM_SHARED((ALLOC, H), f32),
        pltpu.VMEM_SHARED((ALLOC,), f32),
        pltpu.VMEM((CHUNK,), i32),
        pltpu.VMEM((CHUNK,), i32),
        pltpu.VMEM((S, 128), i32),
        pltpu.VMEM((S, 128, H), f32),
        pltpu.VMEM((16,), i32),
        pltpu.VMEM((128,), f32),
        pltpu.SemaphoreType.DMA,
        pltpu.SemaphoreType.DMA,
        pltpu.SemaphoreType.DMA,
    ]


def _make_sc_layer1():
    phases = [
        (0, 0, 8, RANGE, STRIPE, ALLOC),
        (1, 1, 8, RANGE, STRIPE, ALLOC),
        (2, 2, 4, RANGE, STRIPE, ALLOC),
        (2, 3, 2, RANGE_D, STRIPE_D, ALLOC_D),
    ]
    body = _sc_stream_kernel(phases, do_cnt=True, num_tables=3)
    f32 = jnp.float32
    out_type = [
        jax.ShapeDtypeStruct((8, ALLOC, H), f32),
        jax.ShapeDtypeStruct((8 * ALLOC,), f32),
        jax.ShapeDtypeStruct((8, ALLOC, H), f32),
        jax.ShapeDtypeStruct((8 * ALLOC,), f32),
        jax.ShapeDtypeStruct((4, ALLOC, H), f32),
        jax.ShapeDtypeStruct((4 * ALLOC,), f32),
        jax.ShapeDtypeStruct((2, ALLOC_D, H), f32),
        jax.ShapeDtypeStruct((2 * ALLOC_D,), f32),
    ]
    return pl.kernel(
        body,
        out_type=out_type,
        mesh=plsc.VectorSubcoreMesh(core_axis_name="c", subcore_axis_name="s"),
        scratch_types=_stream_scratch(),
        name="sc_sage_layer1",
        compiler_params=pltpu.CompilerParams(
            needs_layout_passes=False, use_tc_tiling_on_sc=False),
    )


def _make_sc_layer2():
    phases = [
        (0, 0, 8, RANGE, STRIPE, ALLOC),
        (1, 1, 8, RANGE, STRIPE, ALLOC),
    ]
    body = _sc_stream_kernel(phases, do_cnt=False, num_tables=2)
    f32 = jnp.float32
    out_type = [
        jax.ShapeDtypeStruct((8, ALLOC, H), f32),
        jax.ShapeDtypeStruct((8, ALLOC, H), f32),
    ]
    return pl.kernel(
        body,
        out_type=out_type,
        mesh=plsc.VectorSubcoreMesh(core_axis_name="c", subcore_axis_name="s"),
        scratch_types=_stream_scratch(),
        name="sc_sage_layer2",
        compiler_params=pltpu.CompilerParams(
            needs_layout_passes=False, use_tc_tiling_on_sc=False),
    )


_DOT = functools.partial(
    jnp.dot, preferred_element_type=jnp.float32, precision=lax.Precision.DEFAULT)


def _proj_body(x_ref, w_ref, b_ref, o_ref, *, relu):
    o = _DOT(x_ref[...], w_ref[...]) + b_ref[...]
    o_ref[...] = jnp.maximum(o, 0.0) if relu else o


def _proj(x, w, b, relu, bn=1000):
    n, d = x.shape
    h = w.shape[1]
    grid = n // bn
    return pl.pallas_call(
        functools.partial(_proj_body, relu=relu),
        grid=(grid,),
        in_specs=[
            pl.BlockSpec((bn, d), lambda i: (i, 0)),
            pl.BlockSpec((d, h), lambda i: (0, 0)),
            pl.BlockSpec((1, h), lambda i: (0, 0)),
        ],
        out_specs=pl.BlockSpec((bn, h), lambda i: (i, 0)),
        out_shape=jax.ShapeDtypeStruct((n, h), jnp.float32),
    )(x, w, b)


def _inv_body(c_ref, o_ref):
    o_ref[...] = (1.0 / jnp.maximum(c_ref[...], 1.0)).T


def _inv_transpose(cnt):
    r, alloc = cnt.shape
    return pl.pallas_call(
        _inv_body,
        out_shape=jax.ShapeDtypeStruct((alloc, r), jnp.float32),
    )(cnt)


def _col(mat, r, nr):
    out = mat[:, 0]
    for j in range(1, nr):
        out = jnp.where(r == j, mat[:, j], out)
    return out


def _combine_tx1_body(aA_ref, iA_ref, aB_ref, iB_ref, hx_ref, wA_ref, wB_ref,
                      wRA_ref, wRB_ref, bias_ref, o_ref, *, per):
    r = pl.program_id(0) // per
    ia = _col(iA_ref[...], r, 8)
    ib = _col(iB_ref[...], r, 8)
    o = (_DOT(aA_ref[0] * ia[:, None], wA_ref[...])
         + _DOT(aB_ref[0] * ib[:, None], wB_ref[...])
         + _DOT(hx_ref[...], wRA_ref[...]) + _DOT(hx_ref[...], wRB_ref[...])
         + bias_ref[...])
    o_ref[...] = jnp.maximum(o, 0.0)


def _combine_tx1(aA, iA, aB, iB, hx, wA, wB, wRA, wRB, bias, bn=800):
    grid = N_TX // bn
    per = RANGE // bn
    acc_spec = pl.BlockSpec((1, bn, H), lambda i: (i // per, i % per, 0))
    inv_spec = pl.BlockSpec((bn, 8), lambda i: (i % per, 0))
    w_spec = pl.BlockSpec((H, H), lambda i: (0, 0))
    return pl.pallas_call(
        functools.partial(_combine_tx1_body, per=per),
        grid=(grid,),
        in_specs=[acc_spec, inv_spec, acc_spec, inv_spec,
                  pl.BlockSpec((bn, H), lambda i: (i, 0)),
                  w_spec, w_spec, w_spec, w_spec,
                  pl.BlockSpec((1, H), lambda i: (0, 0))],
        out_specs=pl.BlockSpec((bn, H), lambda i: (i, 0)),
        out_shape=jax.ShapeDtypeStruct((N_TX, H), jnp.float32),
    )(aA, iA, aB, iB, hx, wA, wB, wRA, wRB, bias)


def _combine_small_body(a_ref, i_ref, hd_ref, wl_ref, wr_ref, bias_ref, o_ref,
                        *, per, nr):
    r = pl.program_id(0) // per
    inv = _col(i_ref[...], r, nr)
    o = (_DOT(a_ref[0] * inv[:, None], wl_ref[...])
         + _DOT(hd_ref[...], wr_ref[...]) + bias_ref[...])
    o_ref[...] = jnp.maximum(o, 0.0)


def _combine_small(a, inv, hd, wl, wr, bias, n, nr, range_rows, bn):
    grid = n // bn
    per = range_rows // bn
    acc_spec = pl.BlockSpec((1, bn, H), lambda i: (i // per, i % per, 0))
    inv_spec = pl.BlockSpec((bn, nr), lambda i: (i % per, 0))
    w_spec = pl.BlockSpec((H, H), lambda i: (0, 0))
    return pl.pallas_call(
        functools.partial(_combine_small_body, per=per, nr=nr),
        grid=(grid,),
        in_specs=[acc_spec, inv_spec,
                  pl.BlockSpec((bn, H), lambda i: (i, 0)),
                  w_spec, w_spec, pl.BlockSpec((1, H), lambda i: (0, 0))],
        out_specs=pl.BlockSpec((bn, H), lambda i: (i, 0)),
        out_shape=jax.ShapeDtypeStruct((n, H), jnp.float32),
    )(a, inv, hd, wl, wr, bias)


def _combine_tx2_head_body(aA_ref, iA_ref, aB_ref, iB_ref, hx_ref, wA_ref,
                           wB_ref, wRA_ref, wRB_ref, bias_ref, wh1_ref,
                           bh1_ref, wh2_ref, bh2_ref, o_ref, *, per):
    r = pl.program_id(0) // per
    ia = _col(iA_ref[...], r, 8)
    ib = _col(iB_ref[...], r, 8)
    o = (_DOT(aA_ref[0] * ia[:, None], wA_ref[...])
         + _DOT(aB_ref[0] * ib[:, None], wB_ref[...])
         + _DOT(hx_ref[...], wRA_ref[...]) + _DOT(hx_ref[...], wRB_ref[...])
         + bias_ref[...])
    tx = jnp.maximum(o, 0.0)
    z = jnp.maximum(_DOT(tx, wh1_ref[...]) + bh1_ref[...], 0.0)
    o_ref[...] = _DOT(z, wh2_ref[...]) + bh2_ref[...]


def _combine_tx2_head(aA, iA, aB, iB, hx, wA, wB, wRA, wRB, bias, wh1, bh1,
                      wh2, bh2, bn=800):
    grid = N_TX // bn
    per = RANGE // bn
    acc_spec = pl.BlockSpec((1, bn, H), lambda i: (i // per, i % per, 0))
    inv_spec = pl.BlockSpec((bn, 8), lambda i: (i % per, 0))
    w_spec = pl.BlockSpec((H, H), lambda i: (0, 0))
    v_spec = pl.BlockSpec((1, H), lambda i: (0, 0))
    return pl.pallas_call(
        functools.partial(_combine_tx2_head_body, per=per),
        grid=(grid,),
        in_specs=[acc_spec, inv_spec, acc_spec, inv_spec,
                  pl.BlockSpec((bn, H), lambda i: (i, 0)),
                  w_spec, w_spec, w_spec, w_spec, v_spec,
                  w_spec, v_spec,
                  pl.BlockSpec((H, 1), lambda i: (0, 0)),
                  pl.BlockSpec((1, 1), lambda i: (0, 0))],
        out_specs=pl.BlockSpec((bn, 1), lambda i: (i, 0)),
        out_shape=jax.ShapeDtypeStruct((N_TX, 1), jnp.float32),
    )(aA, iA, aB, iB, hx, wA, wB, wRA, wRB, bias, wh1, bh1, wh2, bh2)


def _pad_edges(ei):
    src = jnp.concatenate(
        [ei[0], jnp.zeros((E_PAD - E,), jnp.int32)])
    dst = jnp.concatenate(
        [ei[1], jnp.full((E_PAD - E,), 1 << 30, jnp.int32)])
    return src, dst


def kernel(x_tx, ids_user, ids_merchant, ei_tpu, ei_urt, ei_tpm, ei_mrt, params):
    p = params
    f32 = jnp.float32

    sA, dA = _pad_edges(ei_urt)
    sB, dB = _pad_edges(ei_mrt)
    sC, dC = _pad_edges(ei_tpu)
    sD, dD = _pad_edges(ei_tpm)

    h_tx = _proj(x_tx, p['W_tx'].T, p['b_tx'].reshape(1, H), relu=True)
    h_u = _proj(p['emb_user'], p['Wp_user'].T, p['bp_user'].reshape(1, H),
                relu=False)
    h_m = _proj(p['emb_merchant'], p['Wp_merchant'].T,
                p['bp_merchant'].reshape(1, H), relu=False)

    z2d = jnp.zeros((ALLOC, H), f32)
    z1d = jnp.zeros((ALLOC,), f32)

    part = _make_sc_partition()
    (psA, pdA, pgA, psB, pdB, pgB, psC, pdC, pgC, psD, pdD, pgD) = part(
        sA, dA, sB, dB, sC, dC, sD, dD)

    sc1 = _make_sc_layer1()
    accA, cntA, accB, cntB, accC, cntC, accD, cntD = sc1(
        h_u, h_m, h_tx, z2d, z1d,
        psA, pdA, pgA, psB, pdB, pgB, psC, pdC, pgC, psD, pdD, pgD)

    invA = _inv_transpose(cntA.reshape(8, ALLOC))
    invB = _inv_transpose(cntB.reshape(8, ALLOC))
    invC = _inv_transpose(cntC.reshape(4, ALLOC))
    invD = _inv_transpose(cntD.reshape(2, ALLOC_D))

    L1 = p['convs'][0]
    wA1 = L1['urt']['Wl'].T
    wB1 = L1['mrt']['Wl'].T
    bias1 = (L1['urt']['bl'] + L1['mrt']['bl']).reshape(1, H)
    h_tx2 = _combine_tx1(accA, invA, accB, invB, h_tx, wA1, wB1,
                         L1['urt']['Wr'].T, L1['mrt']['Wr'].T, bias1)
    h_u2 = _combine_small(accC, invC, h_u, L1['tpu']['Wl'].T,
                          L1['tpu']['Wr'].T, L1['tpu']['bl'].reshape(1, H),
                          N_U, 4, RANGE, 400)
    h_m2 = _combine_small(accD, invD, h_m, L1['tpm']['Wl'].T,
                          L1['tpm']['Wr'].T, L1['tpm']['bl'].reshape(1, H),
                          N_M, 2, RANGE_D, 1000)

    sc2 = _make_sc_layer2()
    accA2, accB2 = sc2(h_u2, h_m2, z2d, z1d, psA, pdA, pgA, psB, pdB, pgB)

    L2 = p['convs'][1]
    wA2 = L2['urt']['Wl'].T
    wB2 = L2['mrt']['Wl'].T
    bias2 = (L2['urt']['bl'] + L2['mrt']['bl']).reshape(1, H)
    logits = _combine_tx2_head(
        accA2, invA, accB2, invB, h_tx2, wA2, wB2,
        L2['urt']['Wr'].T, L2['mrt']['Wr'].T, bias2,
        p['Wh1'].T, p['bh1'].reshape(1, H), p['Wh2'].reshape(H, 1),
        p['bh2'].reshape(1, 1).astype(f32))
    return logits.reshape(-1)


# speedup vs baseline: 1.2750x; 1.2750x over previous
import functools

import jax
import jax.numpy as jnp
from jax import lax
from jax.experimental import pallas as pl
from jax.experimental.pallas import tpu as pltpu
from jax.experimental.pallas import tpu_sc as plsc

N_TX, N_U, N_M = 100000, 50000, 10000
E = 300000
D_IN, H = 128, 64

NC, NS = 2, 16
BLK = 1920
BPT = 10
SLICE = BLK * BPT
E_PAD = SLICE * NS + BLK
S = 6
CS = S * 128
RANGE = 12800
STRIPE = 808
ALLOC = STRIPE * NS
RANGE_D = 5000
STRIPE_D = 320
ALLOC_D = STRIPE_D * NS
UNROLL = 4
DUMP = 832
CS_ALLOC = DUMP + 16


def _sc_segment_kernel(phases, do_cnt, num_tables):
    n_edges = 2 * len(phases)
    n_out = len(phases) * (2 if do_cnt else 1)

    def body(*refs):
        nt = num_tables + 2
        tabs = refs[:num_tables]
        z2d, z1d = refs[num_tables:nt]
        edges = refs[nt:nt + n_edges]
        outs = refs[nt + n_edges:nt + n_edges + n_out]
        (acc_sp, cnt_sp, ebs0, ebd0, ebs1, ebd1, cs_src, cs_dst, di, rows,
         ones, sem_e, sem_g, sem_s, sem_c) = refs[nt + n_edges + n_out:]

        c = lax.axis_index("c")
        s = lax.axis_index("s")

        zero16i = jnp.zeros((16,), jnp.int32)
        one16f = jnp.ones((16,), jnp.float32)

        for u in range(8):
            ones[pl.ds(u * 16, 16)] = one16f

        def wait_e():
            pltpu.make_async_copy(edges[0].at[pl.ds(0, BLK)], ebs0, sem_e).wait()

        def flush(table, g, use_cnt):
            @pl.when(g > 0)
            def _():
                for _ in range(S):
                    pltpu.make_async_copy(
                        rows.at[0], acc_sp.at[di.at[0]], sem_s).wait()
                if use_cnt:
                    for _ in range(S):
                        pltpu.make_async_copy(
                            ones, cnt_sp.at[di.at[0]], sem_c).wait()
            for k in range(S):
                for u in range(8):
                    di[k, pl.ds(u * 16, 16)] = cs_dst[pl.ds(k * 128 + u * 16, 16)]
            for k in range(S):
                pltpu.async_copy(
                    table.at[cs_src.at[pl.ds(k * 128, 128)]], rows.at[k], sem_g)
            for k in range(S):
                pltpu.make_async_copy(
                    table.at[cs_src.at[pl.ds(k * 128, 128)]], rows.at[k], sem_g).wait()
            for k in range(S):
                pltpu.async_copy(rows.at[k], acc_sp.at[di.at[k]], sem_s, add=True)
                if use_cnt:
                    pltpu.async_copy(ones, cnt_sp.at[di.at[k]], sem_c, add=True)
            for u in range(UNROLL):
                v = cs_src[pl.ds(CS + u * 16, 16)]
                cs_src[pl.ds(u * 16, 16)] = v
                w = cs_dst[pl.ds(CS + u * 16, 16)]
                cs_dst[pl.ds(u * 16, 16)] = w

        def scan(src_h, dst_h, table, lo, range_rows, alloc, use_cnt):
            hi = lo + range_rows
            trash16 = jnp.full((16,), alloc - 1, jnp.int32)
            ebase = s * SLICE

            def stage(b, bs, bd):
                pltpu.async_copy(src_h.at[pl.ds(ebase + b * BLK, BLK)], bs, sem_e)
                pltpu.async_copy(dst_h.at[pl.ds(ebase + b * BLK, BLK)], bd, sem_e)

            def filter_block(ebs, ebd, carry):
                def it(i, carry):
                    nfill, g = carry
                    ds_ = [ebd[pl.ds((i * UNROLL + u) * 16, 16)]
                           for u in range(UNROLL)]
                    ss_ = [ebs[pl.ds((i * UNROLL + u) * 16, 16)]
                           for u in range(UNROLL)]
                    ms_ = [(d >= lo) & (d < hi) for d in ds_]
                    cums = [jnp.cumsum(m.astype(jnp.int32)) for m in ms_]
                    ns = [jnp.max(cu) for cu in cums]
                    for u in range(UNROLL):
                        pos = jnp.where(ms_[u], nfill + cums[u] - 1, DUMP)
                        plsc.store_scatter(cs_dst, [pos], ds_[u] - lo)
                        plsc.store_scatter(cs_src, [pos], ss_[u])
                        nfill = nfill + ns[u]
                    full = nfill >= CS

                    @pl.when(full)
                    def _():
                        flush(table, g, use_cnt)
                    nfill = jnp.where(full, nfill - CS, nfill)
                    g = g + full.astype(jnp.int32)
                    return (nfill, g)
                return lax.fori_loop(0, BLK // (16 * UNROLL), it, carry)

            stage(0, ebs0, ebd0)

            def outer(o, carry):
                b0 = 2 * o
                wait_e()
                wait_e()
                stage(b0 + 1, ebs1, ebd1)
                carry = filter_block(ebs0, ebd0, carry)
                wait_e()
                wait_e()
                stage(b0 + 2, ebs0, ebd0)
                carry = filter_block(ebs1, ebd1, carry)
                return carry

            nfill, g = lax.fori_loop(0, BPT // 2, outer, (0, 0))

            trips = (CS - nfill + 15) // 16

            def padit(j, _):
                cs_dst[pl.ds(nfill + j * 16, 16)] = trash16
                cs_src[pl.ds(nfill + j * 16, 16)] = zero16i
                return 0
            lax.fori_loop(0, trips, padit, 0)
            flush(table, g, use_cnt)
            for _ in range(S):
                pltpu.make_async_copy(rows.at[0], acc_sp.at[di.at[0]], sem_s).wait()
            if use_cnt:
                for _ in range(S):
                    pltpu.make_async_copy(ones, cnt_sp.at[di.at[0]], sem_c).wait()
            wait_e()
            wait_e()

        out_i = 0
        for p_i, (tab_i, src_i, dst_i, n_ranges, range_rows, stripe, alloc) in enumerate(phases):
            table = tabs[tab_i]
            src_h, dst_h = edges[src_i], edges[dst_i]
            acc_out = outs[out_i]
            cnt_out = outs[out_i + 1] if do_cnt else None
            out_i += 2 if do_cnt else 1
            half = n_ranges // 2

            def range_body(q, _):
                r = c * half + q
                lo = r * range_rows
                plsc.subcore_barrier()
                pltpu.sync_copy(
                    z2d.at[pl.ds(s * stripe, stripe)],
                    acc_sp.at[pl.ds(s * stripe, stripe)])
                if do_cnt:
                    pltpu.sync_copy(
                        z1d.at[pl.ds(s * stripe, stripe)],
                        cnt_sp.at[pl.ds(s * stripe, stripe)])
                plsc.subcore_barrier()
                scan(src_h, dst_h, table, lo, range_rows, alloc, do_cnt)
                plsc.subcore_barrier()
                pltpu.sync_copy(
                    acc_sp.at[pl.ds(s * stripe, stripe)],
                    acc_out.at[r, pl.ds(s * stripe, stripe)])
                if do_cnt:
                    pltpu.sync_copy(
                        cnt_sp.at[pl.ds(s * stripe, stripe)],
                        cnt_out.at[pl.ds(r * alloc + s * stripe, stripe)])
                return 0

            lax.fori_loop(0, half, range_body, 0)

    return body


def _make_sc_layer1():
    phases = [
        (0, 0, 1, 8, RANGE, STRIPE, ALLOC),
        (1, 2, 3, 8, RANGE, STRIPE, ALLOC),
        (2, 4, 5, 4, RANGE, STRIPE, ALLOC),
        (2, 6, 7, 2, RANGE_D, STRIPE_D, ALLOC_D),
    ]
    body = _sc_segment_kernel(phases, do_cnt=True, num_tables=3)
    f32 = jnp.float32
    out_type = [
        jax.ShapeDtypeStruct((8, ALLOC, H), f32),
        jax.ShapeDtypeStruct((8 * ALLOC,), f32),
        jax.ShapeDtypeStruct((8, ALLOC, H), f32),
        jax.ShapeDtypeStruct((8 * ALLOC,), f32),
        jax.ShapeDtypeStruct((4, ALLOC, H), f32),
        jax.ShapeDtypeStruct((4 * ALLOC,), f32),
        jax.ShapeDtypeStruct((2, ALLOC_D, H), f32),
        jax.ShapeDtypeStruct((2 * ALLOC_D,), f32),
    ]
    return pl.kernel(
        body,
        out_type=out_type,
        mesh=plsc.VectorSubcoreMesh(core_axis_name="c", subcore_axis_name="s"),
        scratch_types=_sc_scratch(),
        name="sc_sage_layer1",
        compiler_params=pltpu.CompilerParams(
            needs_layout_passes=False, use_tc_tiling_on_sc=False),
    )


def _make_sc_layer2():
    phases = [
        (0, 0, 1, 8, RANGE, STRIPE, ALLOC),
        (1, 2, 3, 8, RANGE, STRIPE, ALLOC),
    ]
    body = _sc_segment_kernel(phases, do_cnt=False, num_tables=2)
    f32 = jnp.float32
    out_type = [
        jax.ShapeDtypeStruct((8, ALLOC, H), f32),
        jax.ShapeDtypeStruct((8, ALLOC, H), f32),
    ]
    return pl.kernel(
        body,
        out_type=out_type,
        mesh=plsc.VectorSubcoreMesh(core_axis_name="c", subcore_axis_name="s"),
        scratch_types=_sc_scratch(),
        name="sc_sage_layer2",
        compiler_params=pltpu.CompilerParams(
            needs_layout_passes=False, use_tc_tiling_on_sc=False),
    )


def _sc_scratch():
    f32, i32 = jnp.float32, jnp.int32
    return [
        pltpu.VMEM_SHARED((ALLOC, H), f32),
        pltpu.VMEM_SHARED((ALLOC,), f32),
        pltpu.VMEM((BLK,), i32),
        pltpu.VMEM((BLK,), i32),
        pltpu.VMEM((BLK,), i32),
        pltpu.VMEM((BLK,), i32),
        pltpu.VMEM((CS_ALLOC,), i32),
        pltpu.VMEM((CS_ALLOC,), i32),
        pltpu.VMEM((S, 128), i32),
        pltpu.VMEM((S, 128, H), f32),
        pltpu.VMEM((128,), f32),
        pltpu.SemaphoreType.DMA,
        pltpu.SemaphoreType.DMA,
        pltpu.SemaphoreType.DMA,
        pltpu.SemaphoreType.DMA,
    ]


_DOT = functools.partial(
    jnp.dot, preferred_element_type=jnp.float32, precision=lax.Precision.DEFAULT)


def _proj_body(x_ref, w_ref, b_ref, o_ref, *, relu):
    o = _DOT(x_ref[...], w_ref[...]) + b_ref[...]
    o_ref[...] = jnp.maximum(o, 0.0) if relu else o


def _proj(x, w, b, relu, bn=1000):
    n, d = x.shape
    h = w.shape[1]
    grid = n // bn
    return pl.pallas_call(
        functools.partial(_proj_body, relu=relu),
        grid=(grid,),
        in_specs=[
            pl.BlockSpec((bn, d), lambda i: (i, 0)),
            pl.BlockSpec((d, h), lambda i: (0, 0)),
            pl.BlockSpec((1, h), lambda i: (0, 0)),
        ],
        out_specs=pl.BlockSpec((bn, h), lambda i: (i, 0)),
        out_shape=jax.ShapeDtypeStruct((n, h), jnp.float32),
    )(x, w, b)


def _inv_body(c_ref, o_ref):
    o_ref[...] = (1.0 / jnp.maximum(c_ref[...], 1.0)).T


def _inv_transpose(cnt):
    r, alloc = cnt.shape
    return pl.pallas_call(
        _inv_body,
        out_shape=jax.ShapeDtypeStruct((alloc, r), jnp.float32),
    )(cnt)


def _col(mat, r, nr):
    out = mat[:, 0]
    for j in range(1, nr):
        out = jnp.where(r == j, mat[:, j], out)
    return out


def _combine_tx1_body(aA_ref, iA_ref, aB_ref, iB_ref, hx_ref, wA_ref, wB_ref,
                      wRA_ref, wRB_ref, bias_ref, o_ref, *, per):
    r = pl.program_id(0) // per
    ia = _col(iA_ref[...], r, 8)
    ib = _col(iB_ref[...], r, 8)
    o = (_DOT(aA_ref[0] * ia[:, None], wA_ref[...])
         + _DOT(aB_ref[0] * ib[:, None], wB_ref[...])
         + _DOT(hx_ref[...], wRA_ref[...]) + _DOT(hx_ref[...], wRB_ref[...])
         + bias_ref[...])
    o_ref[...] = jnp.maximum(o, 0.0)


def _combine_tx1(aA, iA, aB, iB, hx, wA, wB, wRA, wRB, bias, bn=800):
    grid = N_TX // bn
    per = RANGE // bn
    acc_spec = pl.BlockSpec((1, bn, H), lambda i: (i // per, i % per, 0))
    inv_spec = pl.BlockSpec((bn, 8), lambda i: (i % per, 0))
    w_spec = pl.BlockSpec((H, H), lambda i: (0, 0))
    return pl.pallas_call(
        functools.partial(_combine_tx1_body, per=per),
        grid=(grid,),
        in_specs=[acc_spec, inv_spec, acc_spec, inv_spec,
                  pl.BlockSpec((bn, H), lambda i: (i, 0)),
                  w_spec, w_spec, w_spec, w_spec,
                  pl.BlockSpec((1, H), lambda i: (0, 0))],
        out_specs=pl.BlockSpec((bn, H), lambda i: (i, 0)),
        out_shape=jax.ShapeDtypeStruct((N_TX, H), jnp.float32),
    )(aA, iA, aB, iB, hx, wA, wB, wRA, wRB, bias)


def _combine_small_body(a_ref, i_ref, hd_ref, wl_ref, wr_ref, bias_ref, o_ref,
                        *, per, nr):
    r = pl.program_id(0) // per
    inv = _col(i_ref[...], r, nr)
    o = (_DOT(a_ref[0] * inv[:, None], wl_ref[...])
         + _DOT(hd_ref[...], wr_ref[...]) + bias_ref[...])
    o_ref[...] = jnp.maximum(o, 0.0)


def _combine_small(a, inv, hd, wl, wr, bias, n, nr, range_rows, bn):
    grid = n // bn
    per = range_rows // bn
    acc_spec = pl.BlockSpec((1, bn, H), lambda i: (i // per, i % per, 0))
    inv_spec = pl.BlockSpec((bn, nr), lambda i: (i % per, 0))
    w_spec = pl.BlockSpec((H, H), lambda i: (0, 0))
    return pl.pallas_call(
        functools.partial(_combine_small_body, per=per, nr=nr),
        grid=(grid,),
        in_specs=[acc_spec, inv_spec,
                  pl.BlockSpec((bn, H), lambda i: (i, 0)),
                  w_spec, w_spec, pl.BlockSpec((1, H), lambda i: (0, 0))],
        out_specs=pl.BlockSpec((bn, H), lambda i: (i, 0)),
        out_shape=jax.ShapeDtypeStruct((n, H), jnp.float32),
    )(a, inv, hd, wl, wr, bias)


def _combine_tx2_head_body(aA_ref, iA_ref, aB_ref, iB_ref, hx_ref, wA_ref,
                           wB_ref, wRA_ref, wRB_ref, bias_ref, wh1_ref,
                           bh1_ref, wh2_ref, bh2_ref, o_ref, *, per):
    r = pl.program_id(0) // per
    ia = _col(iA_ref[...], r, 8)
    ib = _col(iB_ref[...], r, 8)
    o = (_DOT(aA_ref[0] * ia[:, None], wA_ref[...])
         + _DOT(aB_ref[0] * ib[:, None], wB_ref[...])
         + _DOT(hx_ref[...], wRA_ref[...]) + _DOT(hx_ref[...], wRB_ref[...])
         + bias_ref[...])
    tx = jnp.maximum(o, 0.0)
    z = jnp.maximum(_DOT(tx, wh1_ref[...]) + bh1_ref[...], 0.0)
    o_ref[...] = _DOT(z, wh2_ref[...]) + bh2_ref[...]


def _combine_tx2_head(aA, iA, aB, iB, hx, wA, wB, wRA, wRB, bias, wh1, bh1,
                      wh2, bh2, bn=800):
    grid = N_TX // bn
    per = RANGE // bn
    acc_spec = pl.BlockSpec((1, bn, H), lambda i: (i // per, i % per, 0))
    inv_spec = pl.BlockSpec((bn, 8), lambda i: (i % per, 0))
    w_spec = pl.BlockSpec((H, H), lambda i: (0, 0))
    v_spec = pl.BlockSpec((1, H), lambda i: (0, 0))
    return pl.pallas_call(
        functools.partial(_combine_tx2_head_body, per=per),
        grid=(grid,),
        in_specs=[acc_spec, inv_spec, acc_spec, inv_spec,
                  pl.BlockSpec((bn, H), lambda i: (i, 0)),
                  w_spec, w_spec, w_spec, w_spec, v_spec,
                  w_spec, v_spec,
                  pl.BlockSpec((H, 1), lambda i: (0, 0)),
                  pl.BlockSpec((1, 1), lambda i: (0, 0))],
        out_specs=pl.BlockSpec((bn, 1), lambda i: (i, 0)),
        out_shape=jax.ShapeDtypeStruct((N_TX, 1), jnp.float32),
    )(aA, iA, aB, iB, hx, wA, wB, wRA, wRB, bias, wh1, bh1, wh2, bh2)


def _pad_edges(ei):
    src = jnp.concatenate(
        [ei[0], jnp.zeros((E_PAD - E,), jnp.int32)])
    dst = jnp.concatenate(
        [ei[1], jnp.full((E_PAD - E,), 1 << 30, jnp.int32)])
    return src, dst


def kernel(x_tx, ids_user, ids_merchant, ei_tpu, ei_urt, ei_tpm, ei_mrt, params):
    p = params
    f32 = jnp.float32

    sA, dA = _pad_edges(ei_urt)
    sB, dB = _pad_edges(ei_mrt)
    sC, dC = _pad_edges(ei_tpu)
    sD, dD = _pad_edges(ei_tpm)

    h_tx = _proj(x_tx, p['W_tx'].T, p['b_tx'].reshape(1, H), relu=True)
    h_u = _proj(p['emb_user'], p['Wp_user'].T, p['bp_user'].reshape(1, H),
                relu=False)
    h_m = _proj(p['emb_merchant'], p['Wp_merchant'].T,
                p['bp_merchant'].reshape(1, H), relu=False)

    z2d = jnp.zeros((ALLOC, H), f32)
    z1d = jnp.zeros((ALLOC,), f32)

    sc1 = _make_sc_layer1()
    accA, cntA, accB, cntB, accC, cntC, accD, cntD = sc1(
        h_u, h_m, h_tx, z2d, z1d, sA, dA, sB, dB, sC, dC, sD, dD)

    invA = _inv_transpose(cntA.reshape(8, ALLOC))
    invB = _inv_transpose(cntB.reshape(8, ALLOC))
    invC = _inv_transpose(cntC.reshape(4, ALLOC))
    invD = _inv_transpose(cntD.reshape(2, ALLOC_D))

    L1 = p['convs'][0]
    wA1 = L1['urt']['Wl'].T
    wB1 = L1['mrt']['Wl'].T
    bias1 = (L1['urt']['bl'] + L1['mrt']['bl']).reshape(1, H)
    h_tx2 = _combine_tx1(accA, invA, accB, invB, h_tx, wA1, wB1,
                         L1['urt']['Wr'].T, L1['mrt']['Wr'].T, bias1)
    h_u2 = _combine_small(accC, invC, h_u, L1['tpu']['Wl'].T,
                          L1['tpu']['Wr'].T, L1['tpu']['bl'].reshape(1, H),
                          N_U, 4, RANGE, 400)
    h_m2 = _combine_small(accD, invD, h_m, L1['tpm']['Wl'].T,
                          L1['tpm']['Wr'].T, L1['tpm']['bl'].reshape(1, H),
                          N_M, 2, RANGE_D, 1000)

    sc2 = _make_sc_layer2()
    accA2, accB2 = sc2(h_u2, h_m2, z2d, z1d, sA, dA, sB, dB)

    L2 = p['convs'][1]
    wA2 = L2['urt']['Wl'].T
    wB2 = L2['mrt']['Wl'].T
    bias2 = (L2['urt']['bl'] + L2['mrt']['bl']).reshape(1, H)
    logits = _combine_tx2_head(
        accA2, invA, accB2, invB, h_tx2, wA2, wB2,
        L2['urt']['Wr'].T, L2['mrt']['Wr'].T, bias2,
        p['Wh1'].T, p['bh1'].reshape(1, H), p['Wh2'].reshape(H, 1),
        p['bh2'].reshape(1, 1).astype(f32))
    return logits.reshape(-1)
